# baseline (device time: 121526 ns/iter reference)
import jax
import jax.numpy as jnp
from jax import lax
from jax.experimental import pallas as pl
from jax.experimental.pallas import tpu as pltpu

C = 4


def kernel(x):
    m, n = x.shape
    q = m // 4
    ch = q // C
    zp_rows = 688
    yp_rows = 672
    xp_rows = 688
    assert zp_rows + yp_rows + xp_rows == q

    xb = x.astype(jnp.bfloat16)

    def body(x_ref, out_ref, b_ref, sxs, rxs, sys_, rys, szs, rzs):
        my_x = lax.axis_index("x")
        my_y = lax.axis_index("y")
        my_z = lax.axis_index("z")
        sy = lax.rem(my_y, 2)
        sz = lax.rem(my_z, 2)
        partner = (1 - my_x, my_y, my_z)
        ydom = (my_x, my_y + 1 - 2 * sy, my_z)
        zdom = (my_x, my_y, my_z + 1 - 2 * sz)

        o_me = (2 * sy + sz) * q
        o_y = (2 * (1 - sy) + sz) * q
        o_z = (2 * sy + (1 - sz)) * q
        o_d = (2 * (1 - sy) + (1 - sz)) * q

        def mk(rows, ssem, rsem, dev):
            return pltpu.make_async_remote_copy(
                src_ref=b_ref.at[rows, :],
                dst_ref=b_ref.at[rows, :],
                send_sem=ssem,
                recv_sem=rsem,
                device_id=dev,
                device_id_type=pl.DeviceIdType.MESH,
            )

        def add(rows):
            out_ref[rows, :] = x_ref[rows, :] + b_ref[rows, :]

        barrier_sem = pltpu.get_barrier_semaphore()
        for nbr in (partner, ydom, zdom):
            pl.semaphore_signal(
                barrier_sem,
                inc=1,
                device_id=nbr,
                device_id_type=pl.DeviceIdType.MESH,
            )
        pl.semaphore_wait(barrier_sem, 3)

        x_rd = []
        for k in range(C):
            rows = pl.ds(o_me + k * ch, ch)
            r = pltpu.make_async_remote_copy(
                src_ref=x_ref.at[rows, :],
                dst_ref=b_ref.at[rows, :],
                send_sem=sxs.at[k],
                recv_sem=rxs.at[k],
                device_id=partner,
                device_id_type=pl.DeviceIdType.MESH,
            )
            r.start()
            x_rd.append(r)
        rows_xp = pl.ds(o_d + zp_rows + yp_rows, xp_rows)
        r = pltpu.make_async_remote_copy(
            src_ref=x_ref.at[rows_xp, :],
            dst_ref=b_ref.at[rows_xp, :],
            send_sem=sxs.at[C],
            recv_sem=rxs.at[C],
            device_id=partner,
            device_id_type=pl.DeviceIdType.MESH,
        )
        r.start()
        x_rd.append(r)

        yin = [mk(pl.ds(o_y + k * ch, ch), sys_.at[k], rys.at[k], ydom)
               for k in range(C)]
        zin = [mk(pl.ds(o_z + k * ch, ch), szs.at[k], rzs.at[k], zdom)
               for k in range(C)]

        y1, z1 = [], []

        def fwd(k):
            x_rd[k].wait_recv()
            rows = pl.ds(o_me + k * ch, ch)
            f = mk(rows, sys_.at[k], rys.at[k], ydom)
            f.start()
            y1.append(f)
            g = mk(rows, szs.at[k], rzs.at[k], zdom)
            g.start()
            z1.append(g)
            add(rows)

        fwd(0)
        fwd(1)
        yin[0].wait_recv()
        add(pl.ds(o_y, ch))
        zin[0].wait_recv()
        add(pl.ds(o_z, ch))
        fwd(2)
        yin[1].wait_recv()
        add(pl.ds(o_y + ch, ch))
        zpo = mk(pl.ds(o_y, zp_rows), szs.at[C], rzs.at[C], zdom)
        zpo.start()
        zin[1].wait_recv()
        add(pl.ds(o_z + ch, ch))
        fwd(3)
        zin[2].wait_recv()
        add(pl.ds(o_z + 2 * ch, ch))
        ypo = mk(pl.ds(o_z + zp_rows, yp_rows), sys_.at[C], rys.at[C], ydom)
        ypo.start()
        yin[2].wait_recv()
        add(pl.ds(o_y + 2 * ch, ch))
        x_rd[C].wait_recv()
        add(pl.ds(o_d + zp_rows + yp_rows, xp_rows))
        yin[3].wait_recv()
        add(pl.ds(o_y + 3 * ch, ch))
        zin[3].wait_recv()
        add(pl.ds(o_z + 3 * ch, ch))
        mk(pl.ds(o_d + zp_rows, yp_rows), sys_.at[C], rys.at[C],
           ydom).wait_recv()
        add(pl.ds(o_d + zp_rows, yp_rows))
        mk(pl.ds(o_d, zp_rows), szs.at[C], rzs.at[C], zdom).wait_recv()
        add(pl.ds(o_d, zp_rows))

        for r in x_rd + y1 + z1 + [ypo, zpo]:
            r.wait_send()

    return pl.pallas_call(
        body,
        out_shape=jax.ShapeDtypeStruct((m, n), jnp.bfloat16),
        in_specs=[pl.BlockSpec(memory_space=pltpu.VMEM)],
        out_specs=pl.BlockSpec(memory_space=pltpu.VMEM),
        scratch_shapes=[
            pltpu.VMEM((m, n), jnp.bfloat16),
            pltpu.SemaphoreType.DMA((C + 1,)),
            pltpu.SemaphoreType.DMA((C + 1,)),
            pltpu.SemaphoreType.DMA((C + 1,)),
            pltpu.SemaphoreType.DMA((C + 1,)),
            pltpu.SemaphoreType.DMA((C + 1,)),
            pltpu.SemaphoreType.DMA((C + 1,)),
        ],
        compiler_params=pltpu.CompilerParams(
            collective_id=0, vmem_limit_bytes=100 * 1024 * 1024
        ),
    )(xb)
